# speedup vs baseline: 9.1553x; 9.1553x over previous
import functools

import jax
import jax.numpy as jnp
from jax import lax
from jax.experimental import pallas as pl
from jax.experimental.pallas import tpu as pltpu
from jax.experimental.pallas import tpu_sc as plsc

_NC = 2
_NS = 16
_NW = _NC * _NS
_CHUNK = 128
_NBUF = 4


@functools.partial(jax.jit, static_argnums=(2, 3))
def _sc_gather(weight, idx3, n_chunks, d):
    per_w = n_chunks * _CHUNK
    mesh = plsc.VectorSubcoreMesh(core_axis_name="c", subcore_axis_name="s")

    @functools.partial(
        pl.kernel,
        mesh=mesh,
        out_type=jax.ShapeDtypeStruct((_NW * per_w, d), jnp.float32),
        scratch_types=[
            pltpu.VMEM((n_chunks, _CHUNK), jnp.int32),
            pltpu.VMEM((_NBUF, _CHUNK, d), jnp.float32),
            pltpu.SemaphoreType.DMA((_NBUF,)),
            pltpu.SemaphoreType.DMA((_NBUF,)),
        ],
    )
    def body(table_hbm, idx_hbm, out_hbm, idx_v, rows_v, gsem, osem):
        wid = lax.axis_index("s") * _NC + lax.axis_index("c")
        base = wid * per_w
        pltpu.sync_copy(idx_hbm.at[wid], idx_v)

        def start_gather(j, b):
            pltpu.make_async_copy(
                table_hbm.at[idx_v.at[j]], rows_v.at[b], gsem.at[b]
            ).start()

        def wait_gather(b):
            pltpu.make_async_copy(
                table_hbm.at[idx_v.at[0]], rows_v.at[b], gsem.at[b]
            ).wait()

        def start_write(j, b):
            pltpu.make_async_copy(
                rows_v.at[b], out_hbm.at[pl.ds(base + j * _CHUNK, _CHUNK)],
                osem.at[b],
            ).start()

        def wait_write(b):
            pltpu.make_async_copy(
                rows_v.at[b], out_hbm.at[pl.ds(base, _CHUNK)], osem.at[b]
            ).wait()

        n_steps = n_chunks // _NBUF

        for b in range(_NBUF):
            start_gather(jnp.int32(b), b)

        def step(g, carry):
            for b in range(_NBUF):
                j = g * _NBUF + b
                wait_gather(b)
                start_write(j, b)
            for b in range(_NBUF):
                @pl.when(g + 1 < n_steps)
                def _():
                    wait_write(b)
                    start_gather((g + 1) * _NBUF + b, b)
            return carry

        lax.fori_loop(0, n_steps, step, jnp.int32(0))

        for b in range(_NBUF):
            wait_write(b)

    return body(weight, idx3)


def kernel(input_, weight):
    batch, hist = input_.shape
    n_emb, d = weight.shape
    total = batch * hist
    per_w = total // _NW
    n_chunks = per_w // _CHUNK
    idx3 = input_.reshape(_NW, n_chunks, _CHUNK)
    out = _sc_gather(weight, idx3, n_chunks, d)
    return out.reshape(batch, hist, d)


# speedup vs baseline: 9.1769x; 1.0024x over previous
import functools

import jax
import jax.numpy as jnp
from jax import lax
from jax.experimental import pallas as pl
from jax.experimental.pallas import tpu as pltpu
from jax.experimental.pallas import tpu_sc as plsc

_NC = 2
_NS = 16
_NW = _NC * _NS
_CHUNK = 128
_NBUF = 5


@functools.partial(jax.jit, static_argnums=(2, 3))
def _sc_gather(weight, idx3, n_chunks, d):
    per_w = n_chunks * _CHUNK
    mesh = plsc.VectorSubcoreMesh(core_axis_name="c", subcore_axis_name="s")

    @functools.partial(
        pl.kernel,
        mesh=mesh,
        out_type=jax.ShapeDtypeStruct((_NW * per_w, d), jnp.float32),
        scratch_types=[
            pltpu.VMEM((n_chunks, _CHUNK), jnp.int32),
            pltpu.VMEM((_NBUF, _CHUNK, d), jnp.float32),
            pltpu.SemaphoreType.DMA((_NBUF,)),
            pltpu.SemaphoreType.DMA((_NBUF,)),
        ],
    )
    def body(table_hbm, idx_hbm, out_hbm, idx_v, rows_v, gsem, osem):
        wid = lax.axis_index("s") * _NC + lax.axis_index("c")
        base = wid * per_w
        pltpu.sync_copy(idx_hbm.at[wid], idx_v)

        def start_gather(j, b):
            pltpu.make_async_copy(
                table_hbm.at[idx_v.at[j]], rows_v.at[b], gsem.at[b]
            ).start()

        def wait_gather(b):
            pltpu.make_async_copy(
                table_hbm.at[idx_v.at[0]], rows_v.at[b], gsem.at[b]
            ).wait()

        def start_write(j, b):
            pltpu.make_async_copy(
                rows_v.at[b], out_hbm.at[pl.ds(base + j * _CHUNK, _CHUNK)],
                osem.at[b],
            ).start()

        def wait_write(b):
            pltpu.make_async_copy(
                rows_v.at[b], out_hbm.at[pl.ds(base, _CHUNK)], osem.at[b]
            ).wait()

        n_steps = n_chunks // _NBUF

        for b in range(_NBUF):
            start_gather(jnp.int32(b), b)

        def step(g, carry):
            for b in range(_NBUF):
                j = g * _NBUF + b
                wait_gather(b)
                start_write(j, b)
            for b in range(_NBUF):
                @pl.when(g + 1 < n_steps)
                def _():
                    wait_write(b)
                    start_gather((g + 1) * _NBUF + b, b)
            return carry

        lax.fori_loop(0, n_steps, step, jnp.int32(0))

        for b in range(_NBUF):
            wait_write(b)

    return body(weight, idx3)


def kernel(input_, weight):
    batch, hist = input_.shape
    n_emb, d = weight.shape
    total = batch * hist
    per_w = total // _NW
    n_chunks = per_w // _CHUNK
    idx3 = input_.reshape(_NW, n_chunks, _CHUNK)
    out = _sc_gather(weight, idx3, n_chunks, d)
    return out.reshape(batch, hist, d)
